# baseline (speedup 1.0000x reference)
import functools

import jax
import jax.numpy as jnp
from jax import lax
from jax.experimental import pallas as pl
from jax.experimental.pallas import tpu as pltpu
from jax.experimental.pallas import tpu_sc as plsc

N, E, D, H, O = 10000, 160000, 256, 512, 256

NCORE = 2
NSUB = 16
NW = NCORE * NSUB
EPW = E // NW
BB = 40
NB = EPW // BB
NP = 10240
RPT = NP // NSUB
CW = 128
CNTW = 128

BBF = 64
NBF = EPW // BBF
TAIL = EPW - NBF * BBF


@functools.lru_cache(maxsize=None)
def _make_segsum(nc, with_cnt):
    mesh = plsc.VectorSubcoreMesh(core_axis_name="c", subcore_axis_name="s")
    out_type = [jax.ShapeDtypeStruct((NCORE, nc, NP, CW), jnp.float32)]
    if with_cnt:
        out_type.append(
            jax.ShapeDtypeStruct((NCORE, NP, CNTW), jnp.float32))
    scratch = [
        pltpu.VMEM((NBF, BBF), jnp.int32),
        pltpu.VMEM((NBF, BBF), jnp.int32),
        pltpu.VMEM((TAIL,), jnp.int32),
        pltpu.VMEM((TAIL,), jnp.int32),
        pltpu.VMEM((BBF, CW), jnp.float32),
        pltpu.VMEM((BBF, CW), jnp.float32),
        pltpu.VMEM((BBF, CW), jnp.float32),
        pltpu.VMEM((TAIL, CW), jnp.float32),
        pltpu.VMEM_SHARED((NP, CW), jnp.float32),
    ] + [pltpu.SemaphoreType.DMA] * 8
    NWP = 4
    WP = RPT // NWP

    @functools.partial(pl.kernel, mesh=mesh, out_type=out_type,
                       scratch_types=scratch)
    def segsum(table, srcs, dsts, srcs_t, dsts_t, zeros_cw, ones_cw,
               out_rows, *rest):
        if with_cnt:
            out_cnt = rest[0]
            rest = rest[1:]
        (srcv, dstv, srct, dstt, bufa, bufb, bufc, buft, acc,
         ga, gb, gc, sa, sb, sc_, semw, semz) = rest
        bufs = [bufa, bufb, bufc]
        gsems = [ga, gb, gc]
        ssems = [sa, sb, sc_]
        cid = lax.axis_index("c")
        sid = lax.axis_index("s")
        wid = sid * NCORE + cid
        r0 = sid * RPT

        def accp(p):
            return acc.at[pl.ds(r0 + p * WP, WP)]

        def zerop(p):
            return zeros_cw.at[pl.ds(r0 + p * WP, WP)]

        pltpu.sync_copy(dsts.at[wid], dstv)
        pltpu.sync_copy(dsts_t.at[wid], dstt)

        if with_cnt:
            pltpu.sync_copy(ones_cw, bufa)
            pltpu.sync_copy(zeros_cw.at[pl.ds(r0, RPT)],
                            acc.at[pl.ds(r0, RPT)])
            plsc.subcore_barrier()

            def cstep(b, carry):
                pltpu.sync_copy(bufa, acc.at[dstv.at[b]], add=True)
                return carry

            lax.fori_loop(0, NBF, cstep, 0)
            pltpu.sync_copy(bufa.at[pl.ds(0, TAIL)], acc.at[dstt],
                            add=True)
            pltpu.sync_copy(srcs.at[0, wid], srcv)
            pltpu.async_copy(table.at[srcv.at[0]], bufa, ga)
            pltpu.async_copy(table.at[srcv.at[1]], bufb, gb)
            plsc.subcore_barrier()

            def caccp(p):
                return acc.at[pl.ds(r0 + p * WP, WP)]

            def cntoutp(p):
                return out_cnt.at[cid, pl.ds(r0 + p * WP, WP)]

            pltpu.async_copy(caccp(0), cntoutp(0), semw)
            for p in range(NWP):
                pltpu.make_async_copy(caccp(p), cntoutp(p), semw).wait()
                if p + 1 < NWP:
                    pltpu.async_copy(caccp(p + 1), cntoutp(p + 1), semw)
                pltpu.async_copy(zerop(p), accp(p), semz)
            for p in range(NWP):
                pltpu.make_async_copy(zerop(p), accp(p), semz).wait()
        else:
            pltpu.sync_copy(srcs.at[0, wid], srcv)
            pltpu.async_copy(table.at[srcv.at[0]], bufa, ga)
            pltpu.async_copy(table.at[srcv.at[1]], bufb, gb)
            pltpu.sync_copy(zeros_cw.at[pl.ds(r0, RPT)],
                            acc.at[pl.ds(r0, RPT)])

        def fire_g(b, k):
            pltpu.async_copy(table.at[srcv.at[b]], bufs[k], gsems[k])

        def wait_g(b, k):
            pltpu.make_async_copy(table.at[srcv.at[b]], bufs[k],
                                  gsems[k]).wait()

        def fire_s(b, k):
            pltpu.async_copy(bufs[k], acc.at[dstv.at[b]], ssems[k],
                             add=True)

        def wait_s(b, k):
            pltpu.make_async_copy(bufs[k], acc.at[dstv.at[b]],
                                  ssems[k]).wait()

        for c in range(nc):
            pltpu.sync_copy(srcs_t.at[c, wid], srct)
            plsc.subcore_barrier()

            wait_g(0, 0)
            fire_s(0, 0)
            fire_g(2, 2)

            def step(i, carry):
                for k3 in range(3):
                    b = 1 + 3 * i + k3
                    kb = (1 + k3) % 3
                    wait_g(b, kb)
                    fire_s(b, kb)
                    wait_s(b - 1, (kb + 2) % 3)
                    fire_g(b + 2, (kb + 2) % 3)
                return carry

            lax.fori_loop(0, (NBF - 3) // 3, step, 0)
            for b in range(1 + 3 * ((NBF - 3) // 3), NBF):
                kb = b % 3
                wait_g(b, kb)
                fire_s(b, kb)
                wait_s(b - 1, (kb + 2) % 3)
                if b + 2 < NBF:
                    fire_g(b + 2, (kb + 2) % 3)
            pltpu.sync_copy(table.at[srct], buft)
            pltpu.sync_copy(buft, acc.at[dstt], add=True)
            wait_s(NBF - 1, (NBF - 1) % 3)
            if c + 1 < nc:
                pltpu.sync_copy(srcs.at[c + 1, wid], srcv)
                fire_g(0, 0)
                fire_g(1, 1)
            plsc.subcore_barrier()

            def outp(p):
                return out_rows.at[cid, c, pl.ds(r0 + p * WP, WP)]

            pltpu.async_copy(accp(0), outp(0), semw)
            for p in range(NWP):
                pltpu.make_async_copy(accp(p), outp(p), semw).wait()
                if p + 1 < NWP:
                    pltpu.async_copy(accp(p + 1), outp(p + 1), semw)
                if c + 1 < nc:
                    pltpu.async_copy(
                        zeros_cw.at[pl.ds(r0 + p * WP, WP)], accp(p), semz)
            if c + 1 < nc:
                for p in range(NWP):
                    pltpu.make_async_copy(
                        zeros_cw.at[pl.ds(r0 + p * WP, WP)],
                        accp(p), semz).wait()

    return segsum


BN = 512


def _mm(a, b):
    return jax.lax.dot(a.astype(jnp.bfloat16), b,
                       preferred_element_type=jnp.float32)


def _dense1_body(p_ref, c_ref, x_ref, wl_ref, bl_ref, wr_ref, out_ref):
    cnt = c_ref[0, :, 0] + c_ref[1, :, 0]
    inv = 1.0 / jnp.maximum(cnt, 1.0)
    agg = jnp.concatenate(
        [p_ref[0, k] + p_ref[1, k] for k in range(D // CW)], axis=1)
    mean = agg * inv[:, None]
    h = _mm(mean, wl_ref[...]) + bl_ref[0][None, :] + _mm(x_ref[...],
                                                          wr_ref[...])
    h = jnp.maximum(h, 0.0)
    for k in range(H // CW):
        out_ref[k] = h[:, k * CW:(k + 1) * CW]


def _dense2_body(p_ref, c_ref, h_ref, wl_ref, bl_ref, wr_ref,
                 wo_ref, bo_ref, out_ref):
    cnt = c_ref[0, :, 0] + c_ref[1, :, 0]
    inv = 1.0 / jnp.maximum(cnt, 1.0)
    agg = jnp.concatenate(
        [p_ref[0, k] + p_ref[1, k] for k in range(H // CW)], axis=1)
    mean = agg * inv[:, None]
    hin = jnp.concatenate([h_ref[k] for k in range(H // CW)], axis=1)
    h2 = _mm(mean, wl_ref[...]) + bl_ref[0][None, :] + _mm(hin, wr_ref[...])
    h2 = jnp.maximum(h2, 0.0)
    out_ref[...] = _mm(h2, wo_ref[...]) + bo_ref[0][None, :]


def _dense1(partials, cnts, x, W_l1, b_l1, W_r1):
    grid = (pl.cdiv(N, BN),)
    return pl.pallas_call(
        _dense1_body,
        grid=grid,
        in_specs=[
            pl.BlockSpec((NCORE, D // CW, BN, CW), lambda i: (0, 0, i, 0)),
            pl.BlockSpec((NCORE, BN, CNTW), lambda i: (0, i, 0)),
            pl.BlockSpec((BN, D), lambda i: (i, 0)),
            pl.BlockSpec((D, H), lambda i: (0, 0)),
            pl.BlockSpec((1, H), lambda i: (0, 0)),
            pl.BlockSpec((D, H), lambda i: (0, 0)),
        ],
        out_specs=pl.BlockSpec((H // CW, BN, CW), lambda i: (0, i, 0)),
        out_shape=jax.ShapeDtypeStruct((H // CW, N, CW), jnp.float32),
    )(partials, cnts, x, W_l1, b_l1.reshape(1, H), W_r1)


def _dense2(partials, cnts, h_chunks, W_l2, b_l2, W_r2, W_lin, b_lin):
    grid = (pl.cdiv(N, BN),)
    return pl.pallas_call(
        _dense2_body,
        grid=grid,
        in_specs=[
            pl.BlockSpec((NCORE, H // CW, BN, CW), lambda i: (0, 0, i, 0)),
            pl.BlockSpec((NCORE, BN, CNTW), lambda i: (0, i, 0)),
            pl.BlockSpec((H // CW, BN, CW), lambda i: (0, i, 0)),
            pl.BlockSpec((H, H), lambda i: (0, 0)),
            pl.BlockSpec((1, H), lambda i: (0, 0)),
            pl.BlockSpec((H, H), lambda i: (0, 0)),
            pl.BlockSpec((H, O), lambda i: (0, 0)),
            pl.BlockSpec((1, O), lambda i: (0, 0)),
        ],
        out_specs=pl.BlockSpec((BN, O), lambda i: (i, 0)),
        out_shape=jax.ShapeDtypeStruct((N, O), jnp.float32),
    )(partials, cnts, h_chunks, W_l2, b_l2.reshape(1, H), W_r2,
      W_lin, b_lin.reshape(1, O))


def kernel(x, edge_index, W_l1, b_l1, W_r1, W_l2, b_l2, W_r2, W_lin, b_lin):
    src = edge_index[0]
    dst = edge_index[1]
    nc1 = D // CW
    nc2 = H // CW
    offs1 = (jnp.arange(nc1, dtype=jnp.int32) * N)[:, None]
    offs2 = (jnp.arange(nc2, dtype=jnp.int32) * N)[:, None]
    srcw = src.reshape(NW, EPW)
    dstw = dst.reshape(NW, EPW)
    src1 = (srcw[None] + offs1[:, :, None])
    src2 = (srcw[None] + offs2[:, :, None])
    src1f = src1[:, :, :NBF * BBF].reshape(nc1, NW, NBF, BBF)
    src1t = src1[:, :, NBF * BBF:]
    src2f = src2[:, :, :NBF * BBF].reshape(nc2, NW, NBF, BBF)
    src2t = src2[:, :, NBF * BBF:]
    dstrf = dstw[:, :NBF * BBF].reshape(NW, NBF, BBF)
    dstrt = dstw[:, NBF * BBF:]

    zeros_cw = jnp.zeros((NP, CW), jnp.float32)
    ones_cw = jnp.ones((BBF, CW), jnp.float32)

    x_chunks = x.reshape(N, nc1, CW).transpose(1, 0, 2).reshape(nc1 * N, CW)
    p1, cnts = _make_segsum(nc1, True)(x_chunks, src1f, dstrf, src1t,
                                       dstrt, zeros_cw, ones_cw)

    bf = jnp.bfloat16
    h_chunks = _dense1(p1, cnts, x.astype(bf), W_l1.astype(bf), b_l1,
                       W_r1.astype(bf))

    (p2,) = _make_segsum(nc2, False)(h_chunks.reshape(nc2 * N, CW), src2f,
                                     dstrf, src2t, dstrt, zeros_cw, ones_cw)

    return _dense2(p2, cnts, h_chunks, W_l2.astype(bf), b_l2,
                   W_r2.astype(bf), W_lin.astype(bf), b_lin)


# speedup vs baseline: 1.0055x; 1.0055x over previous
import functools

import jax
import jax.numpy as jnp
from jax import lax
from jax.experimental import pallas as pl
from jax.experimental.pallas import tpu as pltpu
from jax.experimental.pallas import tpu_sc as plsc

N, E, D, H, O = 10000, 160000, 256, 512, 256

NCORE = 2
NSUB = 16
NW = NCORE * NSUB
EPW = E // NW
BB = 40
NB = EPW // BB
NP = 10240
RPT = NP // NSUB
CW = 128
CNTW = 128

BBF = 128
NBF = EPW // BBF
TAIL = EPW - NBF * BBF


@functools.lru_cache(maxsize=None)
def _make_segsum(nc, with_cnt):
    mesh = plsc.VectorSubcoreMesh(core_axis_name="c", subcore_axis_name="s")
    out_type = [jax.ShapeDtypeStruct((NCORE, nc, NP, CW), jnp.float32)]
    if with_cnt:
        out_type.append(
            jax.ShapeDtypeStruct((NCORE, NP, CNTW), jnp.float32))
    scratch = [
        pltpu.VMEM((NBF, BBF), jnp.int32),
        pltpu.VMEM((NBF, BBF), jnp.int32),
        pltpu.VMEM((TAIL,), jnp.int32),
        pltpu.VMEM((TAIL,), jnp.int32),
        pltpu.VMEM((BBF, CW), jnp.float32),
        pltpu.VMEM((BBF, CW), jnp.float32),
        pltpu.VMEM((TAIL, CW), jnp.float32),
        pltpu.VMEM_SHARED((NP, CW), jnp.float32),
        pltpu.SemaphoreType.DMA,
        pltpu.SemaphoreType.DMA,
        pltpu.SemaphoreType.DMA,
        pltpu.SemaphoreType.DMA,
    ]
    NWP = 4
    WP = RPT // NWP

    @functools.partial(pl.kernel, mesh=mesh, out_type=out_type,
                       scratch_types=scratch)
    def segsum(table, srcs, dsts, srcs_t, dsts_t, zeros_cw, ones_cw,
               out_rows, *rest):
        if with_cnt:
            out_cnt = rest[0]
            rest = rest[1:]
        (srcv, dstv, srct, dstt, bufa, bufb, buft, acc,
         sema, semb, semw, semz) = rest
        cid = lax.axis_index("c")
        sid = lax.axis_index("s")
        wid = sid * NCORE + cid
        r0 = sid * RPT

        def accp(p):
            return acc.at[pl.ds(r0 + p * WP, WP)]

        def zerop(p):
            return zeros_cw.at[pl.ds(r0 + p * WP, WP)]

        pltpu.sync_copy(dsts.at[wid], dstv)
        pltpu.sync_copy(dsts_t.at[wid], dstt)

        if with_cnt:
            pltpu.sync_copy(ones_cw, bufa)
            pltpu.sync_copy(zeros_cw.at[pl.ds(r0, RPT)],
                            acc.at[pl.ds(r0, RPT)])
            plsc.subcore_barrier()

            def cstep(b, carry):
                pltpu.sync_copy(bufa, acc.at[dstv.at[b]], add=True)
                return carry

            lax.fori_loop(0, NBF, cstep, 0)
            pltpu.sync_copy(bufa.at[pl.ds(0, TAIL)], acc.at[dstt],
                            add=True)
            pltpu.sync_copy(srcs.at[0, wid], srcv)
            pltpu.async_copy(table.at[srcv.at[0]], bufa, sema)
            plsc.subcore_barrier()

            def caccp(p):
                return acc.at[pl.ds(r0 + p * WP, WP)]

            def cntoutp(p):
                return out_cnt.at[cid, pl.ds(r0 + p * WP, WP)]

            pltpu.async_copy(caccp(0), cntoutp(0), semw)
            for p in range(NWP):
                pltpu.make_async_copy(caccp(p), cntoutp(p), semw).wait()
                if p + 1 < NWP:
                    pltpu.async_copy(caccp(p + 1), cntoutp(p + 1), semw)
                pltpu.async_copy(zerop(p), accp(p), semz)
            for p in range(NWP):
                pltpu.make_async_copy(zerop(p), accp(p), semz).wait()
        else:
            pltpu.sync_copy(srcs.at[0, wid], srcv)
            pltpu.async_copy(table.at[srcv.at[0]], bufa, sema)
            pltpu.sync_copy(zeros_cw.at[pl.ds(r0, RPT)],
                            acc.at[pl.ds(r0, RPT)])

        for c in range(nc):
            pltpu.sync_copy(srcs_t.at[c, wid], srct)
            plsc.subcore_barrier()

            def step(i, carry):
                b = 2 * i
                pltpu.async_copy(table.at[srcv.at[b + 1]], bufb, semb)
                pltpu.make_async_copy(table.at[srcv.at[b]],
                                      bufa, sema).wait()
                pltpu.sync_copy(bufa, acc.at[dstv.at[b]], add=True)
                pltpu.async_copy(table.at[srcv.at[b + 2]], bufa, sema)
                pltpu.make_async_copy(table.at[srcv.at[b + 1]],
                                      bufb, semb).wait()
                pltpu.sync_copy(bufb, acc.at[dstv.at[b + 1]], add=True)
                return carry

            lax.fori_loop(0, (NBF - 1) // 2, step, 0)
            pltpu.make_async_copy(table.at[srcv.at[NBF - 1]],
                                  bufa, sema).wait()
            pltpu.sync_copy(bufa, acc.at[dstv.at[NBF - 1]], add=True)
            if c + 1 < nc:
                pltpu.sync_copy(srcs.at[c + 1, wid], srcv)
                pltpu.async_copy(table.at[srcv.at[0]], bufa, sema)
            pltpu.sync_copy(table.at[srct], buft)
            pltpu.sync_copy(buft, acc.at[dstt], add=True)
            plsc.subcore_barrier()

            def outp(p):
                return out_rows.at[cid, c, pl.ds(r0 + p * WP, WP)]

            pltpu.async_copy(accp(0), outp(0), semw)
            for p in range(NWP):
                pltpu.make_async_copy(accp(p), outp(p), semw).wait()
                if p + 1 < NWP:
                    pltpu.async_copy(accp(p + 1), outp(p + 1), semw)
                if c + 1 < nc:
                    pltpu.async_copy(
                        zeros_cw.at[pl.ds(r0 + p * WP, WP)], accp(p), semz)
            if c + 1 < nc:
                for p in range(NWP):
                    pltpu.make_async_copy(
                        zeros_cw.at[pl.ds(r0 + p * WP, WP)],
                        accp(p), semz).wait()

    return segsum


BN = 512


def _mm(a, b):
    return jax.lax.dot(a.astype(jnp.bfloat16), b,
                       preferred_element_type=jnp.float32)


def _dense1_body(p_ref, c_ref, x_ref, wl_ref, bl_ref, wr_ref, out_ref):
    cnt = c_ref[0, :, 0] + c_ref[1, :, 0]
    inv = 1.0 / jnp.maximum(cnt, 1.0)
    agg = jnp.concatenate(
        [p_ref[0, k] + p_ref[1, k] for k in range(D // CW)], axis=1)
    mean = agg * inv[:, None]
    h = _mm(mean, wl_ref[...]) + bl_ref[0][None, :] + _mm(x_ref[...],
                                                          wr_ref[...])
    h = jnp.maximum(h, 0.0)
    for k in range(H // CW):
        out_ref[k] = h[:, k * CW:(k + 1) * CW]


def _dense2_body(p_ref, c_ref, h_ref, wl_ref, bl_ref, wr_ref,
                 wo_ref, bo_ref, out_ref):
    cnt = c_ref[0, :, 0] + c_ref[1, :, 0]
    inv = 1.0 / jnp.maximum(cnt, 1.0)
    agg = jnp.concatenate(
        [p_ref[0, k] + p_ref[1, k] for k in range(H // CW)], axis=1)
    mean = agg * inv[:, None]
    hin = jnp.concatenate([h_ref[k] for k in range(H // CW)], axis=1)
    h2 = _mm(mean, wl_ref[...]) + bl_ref[0][None, :] + _mm(hin, wr_ref[...])
    h2 = jnp.maximum(h2, 0.0)
    out_ref[...] = _mm(h2, wo_ref[...]) + bo_ref[0][None, :]


def _dense1(partials, cnts, x, W_l1, b_l1, W_r1):
    grid = (pl.cdiv(N, BN),)
    return pl.pallas_call(
        _dense1_body,
        grid=grid,
        in_specs=[
            pl.BlockSpec((NCORE, D // CW, BN, CW), lambda i: (0, 0, i, 0)),
            pl.BlockSpec((NCORE, BN, CNTW), lambda i: (0, i, 0)),
            pl.BlockSpec((BN, D), lambda i: (i, 0)),
            pl.BlockSpec((D, H), lambda i: (0, 0)),
            pl.BlockSpec((1, H), lambda i: (0, 0)),
            pl.BlockSpec((D, H), lambda i: (0, 0)),
        ],
        out_specs=pl.BlockSpec((H // CW, BN, CW), lambda i: (0, i, 0)),
        out_shape=jax.ShapeDtypeStruct((H // CW, N, CW), jnp.float32),
    )(partials, cnts, x, W_l1, b_l1.reshape(1, H), W_r1)


def _dense2(partials, cnts, h_chunks, W_l2, b_l2, W_r2, W_lin, b_lin):
    grid = (pl.cdiv(N, BN),)
    return pl.pallas_call(
        _dense2_body,
        grid=grid,
        in_specs=[
            pl.BlockSpec((NCORE, H // CW, BN, CW), lambda i: (0, 0, i, 0)),
            pl.BlockSpec((NCORE, BN, CNTW), lambda i: (0, i, 0)),
            pl.BlockSpec((H // CW, BN, CW), lambda i: (0, i, 0)),
            pl.BlockSpec((H, H), lambda i: (0, 0)),
            pl.BlockSpec((1, H), lambda i: (0, 0)),
            pl.BlockSpec((H, H), lambda i: (0, 0)),
            pl.BlockSpec((H, O), lambda i: (0, 0)),
            pl.BlockSpec((1, O), lambda i: (0, 0)),
        ],
        out_specs=pl.BlockSpec((BN, O), lambda i: (i, 0)),
        out_shape=jax.ShapeDtypeStruct((N, O), jnp.float32),
    )(partials, cnts, h_chunks, W_l2, b_l2.reshape(1, H), W_r2,
      W_lin, b_lin.reshape(1, O))


def kernel(x, edge_index, W_l1, b_l1, W_r1, W_l2, b_l2, W_r2, W_lin, b_lin):
    src = edge_index[0]
    dst = edge_index[1]
    nc1 = D // CW
    nc2 = H // CW
    offs1 = (jnp.arange(nc1, dtype=jnp.int32) * N)[:, None]
    offs2 = (jnp.arange(nc2, dtype=jnp.int32) * N)[:, None]
    srcw = src.reshape(NW, EPW)
    dstw = dst.reshape(NW, EPW)
    src1 = (srcw[None] + offs1[:, :, None])
    src2 = (srcw[None] + offs2[:, :, None])
    src1f = src1[:, :, :NBF * BBF].reshape(nc1, NW, NBF, BBF)
    src1t = src1[:, :, NBF * BBF:]
    src2f = src2[:, :, :NBF * BBF].reshape(nc2, NW, NBF, BBF)
    src2t = src2[:, :, NBF * BBF:]
    dstrf = dstw[:, :NBF * BBF].reshape(NW, NBF, BBF)
    dstrt = dstw[:, NBF * BBF:]

    zeros_cw = jnp.zeros((NP, CW), jnp.float32)
    ones_cw = jnp.ones((BBF, CW), jnp.float32)

    x_chunks = x.reshape(N, nc1, CW).transpose(1, 0, 2).reshape(nc1 * N, CW)
    p1, cnts = _make_segsum(nc1, True)(x_chunks, src1f, dstrf, src1t,
                                       dstrt, zeros_cw, ones_cw)

    bf = jnp.bfloat16
    h_chunks = _dense1(p1, cnts, x.astype(bf), W_l1.astype(bf), b_l1,
                       W_r1.astype(bf))

    (p2,) = _make_segsum(nc2, False)(h_chunks.reshape(nc2 * N, CW), src2f,
                                     dstrf, src2t, dstrt, zeros_cw, ones_cw)

    return _dense2(p2, cnts, h_chunks, W_l2.astype(bf), b_l2,
                   W_r2.astype(bf), W_lin.astype(bf), b_lin)
